# Initial kernel scaffold; baseline (speedup 1.0000x reference)
import jax
import jax.numpy as jnp
from jax import lax
from jax.experimental import pallas as pl
from jax.experimental.pallas import tpu as pltpu
from jax.experimental.pallas import tpu_sc as plsc

N = 1048576
D = 64
NUM_ATOM = 100
NUM_BOND = 10
NC = 2
NS = 16
NW = NC * NS
PER_W = N // NW
CHUNK = 128
NCHUNK = PER_W // CHUNK


def _body(atom_ids, bond_ids, atom_table, bond_table, atom_out, bond_out,
          atom_sh, bond_sh, idx_v, rows_v, sem):
    cid = lax.axis_index("c")
    sid = lax.axis_index("s")
    wid = sid * NC + cid

    @pl.when(sid == 0)
    def _stage():
        pltpu.sync_copy(atom_table, atom_sh)
        pltpu.sync_copy(bond_table, bond_sh)

    plsc.subcore_barrier()

    base = wid * PER_W

    def do_table(ids_hbm, sh_table, out_hbm):
        def chunk_body(i, carry):
            off = base + i * CHUNK
            pltpu.sync_copy(ids_hbm.at[pl.ds(off, CHUNK)], idx_v)
            pltpu.async_copy(sh_table.at[idx_v], rows_v, sem).wait()
            pltpu.sync_copy(rows_v, out_hbm.at[pl.ds(off, CHUNK)])
            return carry
        lax.fori_loop(0, NCHUNK, chunk_body, 0)

    do_table(atom_ids, atom_sh, atom_out)
    do_table(bond_ids, bond_sh, bond_out)


@jax.jit
def kernel(atom_ids, bond_ids, atom_table, bond_table):
    mesh = plsc.VectorSubcoreMesh(core_axis_name="c", subcore_axis_name="s")
    out_type = (
        jax.ShapeDtypeStruct((N, D), jnp.float32),
        jax.ShapeDtypeStruct((N, D), jnp.float32),
    )
    run = pl.kernel(
        _body,
        out_type=out_type,
        mesh=mesh,
        scratch_types=[
            pltpu.VMEM_SHARED((NUM_ATOM, D), jnp.float32),
            pltpu.VMEM_SHARED((NUM_BOND, D), jnp.float32),
            pltpu.VMEM((CHUNK,), jnp.int32),
            pltpu.VMEM((CHUNK, D), jnp.float32),
            pltpu.SemaphoreType.DMA,
        ],
    )
    return run(atom_ids.astype(jnp.int32), bond_ids.astype(jnp.int32),
               atom_table, bond_table)


# speedup vs baseline: 1.4066x; 1.4066x over previous
import jax
import jax.numpy as jnp
from jax import lax
from jax.experimental import pallas as pl
from jax.experimental.pallas import tpu as pltpu
from jax.experimental.pallas import tpu_sc as plsc

N = 1048576
D = 64
NUM_ATOM = 100
NUM_BOND = 10
NC = 2
NS = 16
NW = NC * NS
PER_W = N // NW
CHUNK = 128
NCHUNK = PER_W // CHUNK


def _body(atom_ids, bond_ids, atom_table, bond_table, atom_out, bond_out,
          atom_sh, bond_sh, idx_v, rows_v, sem):
    cid = lax.axis_index("c")
    sid = lax.axis_index("s")
    wid = sid * NC + cid

    base = wid * PER_W

    def do_table(ids_hbm, sh_table, out_hbm):
        def chunk_body(i, carry):
            off = base + i * CHUNK
            pltpu.sync_copy(ids_hbm.at[pl.ds(off, CHUNK)], idx_v)
            pltpu.async_copy(sh_table.at[idx_v], rows_v, sem).wait()
            pltpu.sync_copy(rows_v, out_hbm.at[pl.ds(off, CHUNK)])
            return carry
        lax.fori_loop(0, NCHUNK, chunk_body, 0)

    do_table(atom_ids, atom_table, atom_out)
    do_table(bond_ids, bond_table, bond_out)


@jax.jit
def kernel(atom_ids, bond_ids, atom_table, bond_table):
    mesh = plsc.VectorSubcoreMesh(core_axis_name="c", subcore_axis_name="s")
    out_type = (
        jax.ShapeDtypeStruct((N, D), jnp.float32),
        jax.ShapeDtypeStruct((N, D), jnp.float32),
    )
    run = pl.kernel(
        _body,
        out_type=out_type,
        mesh=mesh,
        scratch_types=[
            pltpu.VMEM_SHARED((NUM_ATOM, D), jnp.float32),
            pltpu.VMEM_SHARED((NUM_BOND, D), jnp.float32),
            pltpu.VMEM((CHUNK,), jnp.int32),
            pltpu.VMEM((CHUNK, D), jnp.float32),
            pltpu.SemaphoreType.DMA,
        ],
        compiler_params=pltpu.CompilerParams(use_tc_tiling_on_sc=False),
    )
    return run(atom_ids.astype(jnp.int32), bond_ids.astype(jnp.int32),
               atom_table, bond_table)


# speedup vs baseline: 4.9961x; 3.5519x over previous
import jax
import jax.numpy as jnp
from jax import lax
from jax.experimental import pallas as pl
from jax.experimental.pallas import tpu as pltpu
from jax.experimental.pallas import tpu_sc as plsc

N = 1048576
D = 64
NUM_ATOM = 100
NUM_BOND = 10
NC = 2
NS = 16
NW = NC * NS
PER_W = N // NW
CHUNK = 128
NCHUNK = PER_W // CHUNK


def _body(atom_ids, bond_ids, atom_table, bond_table, atom_out, bond_out,
          atom_sh, bond_sh, idx_v, rows_v, sem):
    cid = lax.axis_index("c")
    sid = lax.axis_index("s")
    wid = sid * NC + cid

    @pl.when(sid == 0)
    def _stage():
        pltpu.sync_copy(atom_table, atom_sh)
        pltpu.sync_copy(bond_table, bond_sh)

    plsc.subcore_barrier()

    base = wid * PER_W

    def do_table(ids_hbm, sh_table, out_hbm):
        def chunk_body(i, carry):
            off = base + i * CHUNK
            pltpu.sync_copy(ids_hbm.at[pl.ds(off, CHUNK)], idx_v)
            pltpu.async_copy(sh_table.at[idx_v], rows_v, sem).wait()
            pltpu.sync_copy(rows_v, out_hbm.at[pl.ds(off, CHUNK)])
            return carry
        lax.fori_loop(0, NCHUNK, chunk_body, 0)

    do_table(atom_ids, atom_sh, atom_out)
    do_table(bond_ids, bond_sh, bond_out)


@jax.jit
def kernel(atom_ids, bond_ids, atom_table, bond_table):
    mesh = plsc.VectorSubcoreMesh(core_axis_name="c", subcore_axis_name="s")
    out_type = (
        jax.ShapeDtypeStruct((N, D), jnp.float32),
        jax.ShapeDtypeStruct((N, D), jnp.float32),
    )
    run = pl.kernel(
        _body,
        out_type=out_type,
        mesh=mesh,
        scratch_types=[
            pltpu.VMEM_SHARED((NUM_ATOM, D), jnp.float32),
            pltpu.VMEM_SHARED((NUM_BOND, D), jnp.float32),
            pltpu.VMEM((CHUNK,), jnp.int32),
            pltpu.VMEM((CHUNK, D), jnp.float32),
            pltpu.SemaphoreType.DMA,
        ],
        compiler_params=pltpu.CompilerParams(use_tc_tiling_on_sc=False),
    )
    return run(atom_ids.astype(jnp.int32), bond_ids.astype(jnp.int32),
               atom_table, bond_table)


# speedup vs baseline: 6.5931x; 1.3197x over previous
import jax
import jax.numpy as jnp
from jax import lax
from jax.experimental import pallas as pl
from jax.experimental.pallas import tpu as pltpu
from jax.experimental.pallas import tpu_sc as plsc

N = 1048576
D = 64
NUM_ATOM = 100
NUM_BOND = 10
NC = 2
NS = 16
NW = NC * NS
PER_W = N // NW
CHUNK = 256
NB = 4
G = PER_W // CHUNK
assert G % NB == 0


def _body(atom_ids, bond_ids, atom_table, bond_table, atom_out, bond_out,
          atom_sh, bond_sh, idx_v, rows_v, isem, gsem, osem):
    cid = lax.axis_index("c")
    sid = lax.axis_index("s")
    wid = sid * NC + cid

    @pl.when(sid == 0)
    def _stage():
        pltpu.sync_copy(atom_table, atom_sh)
        pltpu.sync_copy(bond_table, bond_sh)

    plsc.subcore_barrier()

    base = wid * PER_W

    def do_table(ids_hbm, sh_table, out_hbm):
        def idx_load(g, b):
            pltpu.make_async_copy(
                ids_hbm.at[pl.ds(base + g * CHUNK, CHUNK)],
                idx_v.at[b], isem.at[b]).start()

        def gather_start(b):
            pltpu.make_async_copy(
                sh_table.at[idx_v.at[b]], rows_v.at[b], gsem.at[b]).start()

        def gather_wait(b):
            pltpu.make_async_copy(
                sh_table.at[idx_v.at[b]], rows_v.at[b], gsem.at[b]).wait()

        def out_start(g, b):
            pltpu.make_async_copy(
                rows_v.at[b],
                out_hbm.at[pl.ds(base + g * CHUNK, CHUNK)],
                osem.at[b]).start()

        def out_drain(b):
            pltpu.make_async_copy(
                out_hbm.at[pl.ds(base, CHUNK)], rows_v.at[b],
                osem.at[b]).wait()

        def idx_wait(g, b):
            pltpu.make_async_copy(
                ids_hbm.at[pl.ds(base + g * CHUNK, CHUNK)],
                idx_v.at[b], isem.at[b]).wait()

        for b in range(NB):
            idx_load(b, b)

        def outer(o, carry):
            for b in range(NB):
                g = o * NB + b
                pb = (b - 1) % NB

                @pl.when(g > 0)
                def _finish_prev():
                    gather_wait(pb)
                    out_start(g - 1, pb)

                    @pl.when(g - 1 + NB < G)
                    def _prefetch():
                        idx_load(g - 1 + NB, pb)

                @pl.when(g >= NB)
                def _rows_free():
                    out_drain(b)

                idx_wait(g, b)
                gather_start(b)
            return carry

        lax.fori_loop(0, G // NB, outer, 0)

        lb = NB - 1
        gather_wait(lb)
        out_start(G - 1, lb)
        for b in range(NB):
            out_drain(b)

    do_table(atom_ids, atom_sh, atom_out)
    do_table(bond_ids, bond_sh, bond_out)


@jax.jit
def kernel(atom_ids, bond_ids, atom_table, bond_table):
    mesh = plsc.VectorSubcoreMesh(core_axis_name="c", subcore_axis_name="s")
    out_type = (
        jax.ShapeDtypeStruct((N, D), jnp.float32),
        jax.ShapeDtypeStruct((N, D), jnp.float32),
    )
    run = pl.kernel(
        _body,
        out_type=out_type,
        mesh=mesh,
        scratch_types=[
            pltpu.VMEM_SHARED((NUM_ATOM, D), jnp.float32),
            pltpu.VMEM_SHARED((NUM_BOND, D), jnp.float32),
            pltpu.VMEM((NB, CHUNK), jnp.int32),
            pltpu.VMEM((NB, CHUNK, D), jnp.float32),
            pltpu.SemaphoreType.DMA((NB,)),
            pltpu.SemaphoreType.DMA((NB,)),
            pltpu.SemaphoreType.DMA((NB,)),
        ],
        compiler_params=pltpu.CompilerParams(use_tc_tiling_on_sc=False),
    )
    return run(atom_ids.astype(jnp.int32), bond_ids.astype(jnp.int32),
               atom_table, bond_table)
